# baseline (speedup 1.0000x reference)
import functools

import numpy as np

import jax
import jax.numpy as jnp
from jax import lax
from jax.experimental import pallas as pl
from jax.experimental.pallas import tpu as pltpu
from jax.experimental.pallas import tpu_sc as plsc


def _sq_threshold(c=np.float32(0.1)):
    lo, hi = np.float32(c * c * 0.5), np.float32(c * c * 2.0)
    for _ in range(64):
        mid = np.float32((lo.astype(np.float64) + hi.astype(np.float64)) / 2)
        if np.sqrt(mid, dtype=np.float32) >= c:
            hi = mid
        else:
            lo = mid
    return float(hi)


_THR2 = _sq_threshold()
_BR = 1024
_BT = 1024


def _nn_body(nrb, br, n, pts_i_ref, pts_j_ref, nn_i_ref, minsq_ref,
             colmin_out_ref, colmin_ref, pjT_ref):
    rb = pl.program_id(1)

    @pl.when(rb == 0)
    def _():
        pjT_ref[...] = jnp.transpose(pts_j_ref[0])

    pi = pts_i_ref[0]
    pj = pjT_ref[...]
    xi, yi, zi = pi[:, 0:1], pi[:, 1:2], pi[:, 2:3]
    xj, yj, zj = pj[0:1, :], pj[1:2, :], pj[2:3, :]
    a2 = xi * xi + yi * yi + zi * zi
    b2 = xj * xj + yj * yj + zj * zj
    ab = xi * xj + yi * yj + zi * zj
    sq = (a2 + b2) - 2.0 * ab

    rowmin = jnp.min(sq, axis=1, keepdims=True)
    colix = lax.broadcasted_iota(jnp.int32, sq.shape, 1)
    rowarg = jnp.min(jnp.where(sq == rowmin, colix, n), axis=1, keepdims=True)
    nn_i_ref[0] = rowarg.astype(jnp.int32)
    minsq_ref[0] = rowmin

    cmin = jnp.min(sq, axis=0, keepdims=True)

    @pl.when(rb == 0)
    def _():
        colmin_ref[...] = jnp.full(colmin_ref.shape, 1e30, jnp.float32)

    colmin_ref[...] = jnp.minimum(cmin, colmin_ref[...])

    @pl.when(rb == nrb - 1)
    def _():
        colmin_out_ref[0] = colmin_ref[...]


def _mutual_nn(pts_i, pts_jT):
    bsz, n, _ = pts_i.shape
    br = _BR
    nrb = n // br
    return pl.pallas_call(
        functools.partial(_nn_body, nrb, br, n),
        grid=(bsz, nrb),
        in_specs=[
            pl.BlockSpec((1, br, 3), lambda b, r: (b, r, 0)),
            pl.BlockSpec((1, n, 3), lambda b, r: (b, 0, 0)),
        ],
        out_specs=[
            pl.BlockSpec((1, br, 1), lambda b, r: (b, r, 0)),
            pl.BlockSpec((1, br, 1), lambda b, r: (b, r, 0)),
            pl.BlockSpec((1, 1, n), lambda b, r: (b, 0, 0)),
        ],
        out_shape=[
            jax.ShapeDtypeStruct((bsz, n, 1), jnp.int32),
            jax.ShapeDtypeStruct((bsz, n, 1), jnp.float32),
            jax.ShapeDtypeStruct((bsz, 1, n), jnp.float32),
        ],
        scratch_shapes=[
            pltpu.VMEM((1, n), jnp.float32),
            pltpu.VMEM((3, n), jnp.float32),
        ],
        compiler_params=pltpu.CompilerParams(
            dimension_semantics=("arbitrary", "arbitrary")),
    )(pts_i, pts_jT)


def _sc_corr(nni, minsq, colmin, ptsf, validf, embf, n, bsz, v, vi_ix, vj_ix):
    bn = bsz * n
    d = embf.shape[-1]
    info = plsc.get_sparse_core_info()
    ncores, nsub = info.num_cores, info.num_subcores
    nw = ncores * nsub
    r = bn // nw
    half = r // 2
    mesh = plsc.VectorSubcoreMesh(core_axis_name="c", subcore_axis_name="s")

    @functools.partial(
        pl.kernel,
        mesh=mesh,
        compiler_params=pltpu.CompilerParams(needs_layout_passes=False),
        out_type=[jax.ShapeDtypeStruct((bn, d), jnp.float32),
                  jax.ShapeDtypeStruct((bn * 8,), jnp.float32)],
        name="sc_corr",
        scratch_types=[
            pltpu.VMEM((r,), jnp.int32),
            pltpu.VMEM((r,), jnp.float32),
            pltpu.VMEM((r * 3,), jnp.float32),
            pltpu.VMEM((r,), jnp.float32),
            pltpu.VMEM((half,), jnp.int32),
            pltpu.VMEM((half,), jnp.int32),
            pltpu.VMEM((n,), jnp.float32),
            pltpu.VMEM((n * 3,), jnp.float32),
            pltpu.VMEM((n,), jnp.float32),
            pltpu.VMEM((half, d), jnp.float32),
            pltpu.VMEM((half, d), jnp.float32),
            pltpu.VMEM((r * 8,), jnp.float32),
            pltpu.SemaphoreType.DMA,
        ],
    )
    def k(nni_h, minsq_h, colmin_h, ptsf_h, validf_h,
          emb_h, embg_h, relm_h,
          nni_v, minsq_v, ptsi_v, vi_v, idx_a, idx_b,
          colmin_t, ptsj_t, vj_t, rows_a, rows_b, relbuf, sem):
        wid = lax.axis_index("c") * nsub + lax.axis_index("s")
        base = wid * r
        b = base // n
        lbase = base - b * n
        goffi = (b * v + vi_ix) * n
        goffj = (b * v + vj_ix) * n
        ii = lax.iota(jnp.int32, 16)

        pltpu.sync_copy(nni_h.at[pl.ds(base, r)], nni_v)
        for g in range(r // 16):
            v_ = nni_v[pl.ds(g * 16, 16)] + goffj
            if g < half // 16:
                idx_a[pl.ds(g * 16, 16)] = v_
            else:
                idx_b[pl.ds(g * 16 - half, 16)] = v_
        cp0 = pltpu.async_copy(emb_h.at[idx_a], rows_a, sem)
        cp1 = pltpu.async_copy(emb_h.at[idx_b], rows_b, sem)

        pltpu.sync_copy(colmin_h.at[pl.ds(b * n, n)], colmin_t)
        pltpu.sync_copy(ptsf_h.at[pl.ds(goffj * 3, n * 3)], ptsj_t)
        pltpu.sync_copy(validf_h.at[pl.ds(goffj, n)], vj_t)
        pltpu.sync_copy(minsq_h.at[pl.ds(base, r)], minsq_v)
        pltpu.sync_copy(ptsf_h.at[pl.ds((goffi + lbase) * 3, r * 3)], ptsi_v)
        pltpu.sync_copy(validf_h.at[pl.ds(goffi + lbase, r)], vi_v)

        zeros16 = jnp.zeros((16,), jnp.float32)
        for g in range(r // 16):
            sl = pl.ds(g * 16, 16)
            idxv = nni_v[sl]
            msq = minsq_v[sl]
            mutual = plsc.load_gather(colmin_t, [idxv]) == msq
            close = msq < _THR2
            vi_ok = vi_v[sl] > 0.5
            vj_ok = plsc.load_gather(vj_t, [idxv]) > 0.5
            vcf = jnp.where(mutual & close & vi_ok & vj_ok,
                            jnp.float32(1.0), jnp.float32(0.0))
            i3 = idxv * 3
            c3 = (g * 16 + ii) * 3
            rx = plsc.load_gather(ptsj_t, [i3]) - plsc.load_gather(ptsi_v, [c3])
            ry = (plsc.load_gather(ptsj_t, [i3 + 1])
                  - plsc.load_gather(ptsi_v, [c3 + 1]))
            rz = (plsc.load_gather(ptsj_t, [i3 + 2])
                  - plsc.load_gather(ptsi_v, [c3 + 2]))
            pos = (g * 16 + ii) * 8
            plsc.store_scatter(relbuf, [pos], rx)
            plsc.store_scatter(relbuf, [pos + 1], ry)
            plsc.store_scatter(relbuf, [pos + 2], rz)
            plsc.store_scatter(relbuf, [pos + 3], vcf)
            plsc.store_scatter(relbuf, [pos + 4], zeros16)
            plsc.store_scatter(relbuf, [pos + 5], zeros16)
            plsc.store_scatter(relbuf, [pos + 6], zeros16)
            plsc.store_scatter(relbuf, [pos + 7], zeros16)

        cp0.wait()
        cp1.wait()
        pltpu.sync_copy(rows_a, embg_h.at[pl.ds(base, half)])
        pltpu.sync_copy(rows_b, embg_h.at[pl.ds(base + half, half)])
        pltpu.sync_copy(relbuf, relm_h.at[pl.ds(base * 8, r * 8)])

    return k(nni, minsq, colmin, ptsf, validf, embf)


def _loss_body(d, emb_i_ref, embg_ref, relm_ref, w1_ref, b1_ref, g_ref,
               beta_ref, w2_ref, b2_ref, sd_ref, sm_ref):
    first = (pl.program_id(0) == 0) & (pl.program_id(1) == 0)
    x = emb_i_ref[0, 0]
    eg = embg_ref[0]
    rel = relm_ref[0]
    w1 = w1_ref[...]
    rel3 = rel[:, 0:3]
    m = rel[:, 3:4]
    norm = jnp.sqrt(jnp.sum(rel3 * rel3, axis=1, keepdims=True))
    dir3 = rel3 / jnp.maximum(norm, 1e-6)
    h = (jnp.dot(x, w1[0:d], preferred_element_type=jnp.float32)
         + jnp.dot(rel3, w1[d:d + 3], preferred_element_type=jnp.float32)
         + jnp.dot(dir3, w1[d + 3:d + 6], preferred_element_type=jnp.float32)
         + b1_ref[...])
    mu = jnp.mean(h, axis=1, keepdims=True)
    var = jnp.mean((h - mu) ** 2, axis=1, keepdims=True)
    hn = (h - mu) / jnp.sqrt(var + 1e-5) * g_ref[...] + beta_ref[...]
    ge = 0.5 * hn * (1.0 + lax.erf(hn / np.float32(np.sqrt(2.0))))
    t = jnp.dot(ge, w2_ref[...], preferred_element_type=jnp.float32) + b2_ref[...]
    dsq = (t - eg) ** 2
    sd = jnp.sum(dsq * m)
    sm = jnp.sum(m)
    sd_ref[...] = jnp.where(first, 0.0, sd_ref[...]) + sd
    sm_ref[...] = jnp.where(first, 0.0, sm_ref[...]) + sm


def _transform_loss(emb4, vi_ix, embg, relm, w1, b1r, gr, betar, w2, b2r):
    bsz, _, n, d = emb4.shape
    bt = _BT
    nt = n // bt
    return pl.pallas_call(
        functools.partial(_loss_body, d),
        grid=(bsz, nt),
        in_specs=[
            pl.BlockSpec((1, 1, bt, d), lambda b, t: (b, vi_ix, t, 0)),
            pl.BlockSpec((1, bt, d), lambda b, t: (b, t, 0)),
            pl.BlockSpec((1, bt, 8), lambda b, t: (b, t, 0)),
            pl.BlockSpec((d + 6, d), lambda b, t: (0, 0)),
            pl.BlockSpec((1, d), lambda b, t: (0, 0)),
            pl.BlockSpec((1, d), lambda b, t: (0, 0)),
            pl.BlockSpec((1, d), lambda b, t: (0, 0)),
            pl.BlockSpec((d, d), lambda b, t: (0, 0)),
            pl.BlockSpec((1, d), lambda b, t: (0, 0)),
        ],
        out_specs=[
            pl.BlockSpec((1, 1), lambda b, t: (0, 0)),
            pl.BlockSpec((1, 1), lambda b, t: (0, 0)),
        ],
        out_shape=[
            jax.ShapeDtypeStruct((1, 1), jnp.float32),
            jax.ShapeDtypeStruct((1, 1), jnp.float32),
        ],
        compiler_params=pltpu.CompilerParams(
            dimension_semantics=("arbitrary", "arbitrary")),
    )(emb4, embg, relm, w1, b1r, gr, betar, w2, b2r)


def kernel(embeddings, pointmaps, valid_masks, W1, b1, ln_g, ln_b, W2, b2):
    bsz, v, n, d = embeddings.shape
    f32 = jnp.float32
    pts = pointmaps.astype(f32)
    valid_f = valid_masks.astype(f32)
    b1r = b1.reshape(1, d)
    gr = ln_g.reshape(1, d)
    betar = ln_b.reshape(1, d)
    b2r = b2.reshape(1, d)

    bn = bsz * n
    total = jnp.float32(0.0)
    npairs = 0
    for i in range(v):
        for j in range(i + 1, v):
            nni3, minsq3, colmin2 = _mutual_nn(pts[:, i], pts[:, j])
            embg, relm = _sc_corr(
                nni3.reshape(bn), minsq3.reshape(bn), colmin2.reshape(bn),
                pts.reshape(bsz * v * n * 3), valid_f.reshape(bsz * v * n),
                embeddings.reshape(bsz * v * n, d), n, bsz, v, i, j)
            sdb, smb = _transform_loss(
                embeddings, i, embg.reshape(bsz, n, d),
                relm.reshape(bsz, n, 8), W1, b1r, gr, betar, W2, b2r)
            total = total + sdb[0, 0] / (smb[0, 0] * d + 1e-6)
            npairs += 1
    return total / npairs


# speedup vs baseline: 1.0158x; 1.0158x over previous
import functools

import numpy as np

import jax
import jax.numpy as jnp
from jax import lax
from jax.experimental import pallas as pl
from jax.experimental.pallas import tpu as pltpu
from jax.experimental.pallas import tpu_sc as plsc


def _sq_threshold(c=np.float32(0.1)):
    lo, hi = np.float32(c * c * 0.5), np.float32(c * c * 2.0)
    for _ in range(64):
        mid = np.float32((lo.astype(np.float64) + hi.astype(np.float64)) / 2)
        if np.sqrt(mid, dtype=np.float32) >= c:
            hi = mid
        else:
            lo = mid
    return float(hi)


_THR2 = _sq_threshold()
_BR = 1024
_BT = 1024


def _nn_body(nrb, br, n, pts_i_ref, pts_jT_ref, nn_i_ref, minsq_ref,
             colmin_out_ref, colmin_ref):
    rb = pl.program_id(1)
    pi = pts_i_ref[0]
    pj = pts_jT_ref[0]
    xi, yi, zi = pi[:, 0:1], pi[:, 1:2], pi[:, 2:3]
    xj, yj, zj = pj[0:1, :], pj[1:2, :], pj[2:3, :]
    a2 = xi * xi + yi * yi + zi * zi
    b2 = xj * xj + yj * yj + zj * zj
    ab = xi * xj + yi * yj + zi * zj
    sq = (a2 + b2) - 2.0 * ab

    rowmin = jnp.min(sq, axis=1, keepdims=True)
    colix = lax.broadcasted_iota(jnp.int32, sq.shape, 1)
    rowarg = jnp.min(jnp.where(sq == rowmin, colix, n), axis=1, keepdims=True)
    nn_i_ref[0] = rowarg.astype(jnp.int32)
    minsq_ref[0] = rowmin

    cmin = jnp.min(sq, axis=0, keepdims=True)

    @pl.when(rb == 0)
    def _():
        colmin_ref[...] = jnp.full(colmin_ref.shape, 1e30, jnp.float32)

    colmin_ref[...] = jnp.minimum(cmin, colmin_ref[...])

    @pl.when(rb == nrb - 1)
    def _():
        colmin_out_ref[0] = colmin_ref[...]


def _mutual_nn(pts_i, pts_jT):
    bsz, n, _ = pts_i.shape
    br = _BR
    nrb = n // br
    return pl.pallas_call(
        functools.partial(_nn_body, nrb, br, n),
        grid=(bsz, nrb),
        in_specs=[
            pl.BlockSpec((1, br, 3), lambda b, r: (b, r, 0)),
            pl.BlockSpec((1, 3, n), lambda b, r: (b, 0, 0)),
        ],
        out_specs=[
            pl.BlockSpec((1, br, 1), lambda b, r: (b, r, 0)),
            pl.BlockSpec((1, br, 1), lambda b, r: (b, r, 0)),
            pl.BlockSpec((1, 1, n), lambda b, r: (b, 0, 0)),
        ],
        out_shape=[
            jax.ShapeDtypeStruct((bsz, n, 1), jnp.int32),
            jax.ShapeDtypeStruct((bsz, n, 1), jnp.float32),
            jax.ShapeDtypeStruct((bsz, 1, n), jnp.float32),
        ],
        scratch_shapes=[
            pltpu.VMEM((1, n), jnp.float32),
        ],
        compiler_params=pltpu.CompilerParams(
            dimension_semantics=("arbitrary", "arbitrary")),
    )(pts_i, pts_jT)


def _sc_corr(nni, minsq, colmin, ptsf, validf, embf, n, bsz, v, vi_ix, vj_ix):
    bn = bsz * n
    d = embf.shape[-1]
    info = plsc.get_sparse_core_info()
    ncores, nsub = info.num_cores, info.num_subcores
    nw = ncores * nsub
    r = bn // nw
    half = r // 2
    mesh = plsc.VectorSubcoreMesh(core_axis_name="c", subcore_axis_name="s")

    @functools.partial(
        pl.kernel,
        mesh=mesh,
        compiler_params=pltpu.CompilerParams(needs_layout_passes=False),
        out_type=[jax.ShapeDtypeStruct((bn, d), jnp.float32),
                  jax.ShapeDtypeStruct((bn * 8,), jnp.float32)],
        name="sc_corr",
        scratch_types=[
            pltpu.VMEM((r,), jnp.int32),
            pltpu.VMEM((r,), jnp.float32),
            pltpu.VMEM((r * 3,), jnp.float32),
            pltpu.VMEM((r,), jnp.float32),
            pltpu.VMEM((half,), jnp.int32),
            pltpu.VMEM((half,), jnp.int32),
            pltpu.VMEM((n,), jnp.float32),
            pltpu.VMEM((n * 3,), jnp.float32),
            pltpu.VMEM((n,), jnp.float32),
            pltpu.VMEM((half, d), jnp.float32),
            pltpu.VMEM((half, d), jnp.float32),
            pltpu.VMEM((r * 8,), jnp.float32),
            pltpu.SemaphoreType.DMA,
        ],
    )
    def k(nni_h, minsq_h, colmin_h, ptsf_h, validf_h,
          emb_h, embg_h, relm_h,
          nni_v, minsq_v, ptsi_v, vi_v, idx_a, idx_b,
          colmin_t, ptsj_t, vj_t, rows_a, rows_b, relbuf, sem):
        wid = lax.axis_index("c") * nsub + lax.axis_index("s")
        base = wid * r
        b = base // n
        lbase = base - b * n
        goffi = (b * v + vi_ix) * n
        goffj = (b * v + vj_ix) * n
        ii = lax.iota(jnp.int32, 16)

        pltpu.sync_copy(nni_h.at[pl.ds(base, r)], nni_v)
        for g in range(r // 16):
            v_ = nni_v[pl.ds(g * 16, 16)] + goffj
            if g < half // 16:
                idx_a[pl.ds(g * 16, 16)] = v_
            else:
                idx_b[pl.ds(g * 16 - half, 16)] = v_
        cp0 = pltpu.async_copy(emb_h.at[idx_a], rows_a, sem)
        cp1 = pltpu.async_copy(emb_h.at[idx_b], rows_b, sem)

        pltpu.sync_copy(colmin_h.at[pl.ds(b * n, n)], colmin_t)
        pltpu.sync_copy(ptsf_h.at[pl.ds(goffj * 3, n * 3)], ptsj_t)
        pltpu.sync_copy(validf_h.at[pl.ds(goffj, n)], vj_t)
        pltpu.sync_copy(minsq_h.at[pl.ds(base, r)], minsq_v)
        pltpu.sync_copy(ptsf_h.at[pl.ds((goffi + lbase) * 3, r * 3)], ptsi_v)
        pltpu.sync_copy(validf_h.at[pl.ds(goffi + lbase, r)], vi_v)

        zeros16 = jnp.zeros((16,), jnp.float32)
        for g in range(r // 16):
            sl = pl.ds(g * 16, 16)
            idxv = nni_v[sl]
            msq = minsq_v[sl]
            mutual = plsc.load_gather(colmin_t, [idxv]) == msq
            close = msq < _THR2
            vi_ok = vi_v[sl] > 0.5
            vj_ok = plsc.load_gather(vj_t, [idxv]) > 0.5
            vcf = jnp.where(mutual & close & vi_ok & vj_ok,
                            jnp.float32(1.0), jnp.float32(0.0))
            i3 = idxv * 3
            c3 = (g * 16 + ii) * 3
            rx = plsc.load_gather(ptsj_t, [i3]) - plsc.load_gather(ptsi_v, [c3])
            ry = (plsc.load_gather(ptsj_t, [i3 + 1])
                  - plsc.load_gather(ptsi_v, [c3 + 1]))
            rz = (plsc.load_gather(ptsj_t, [i3 + 2])
                  - plsc.load_gather(ptsi_v, [c3 + 2]))
            pos = (g * 16 + ii) * 8
            plsc.store_scatter(relbuf, [pos], rx)
            plsc.store_scatter(relbuf, [pos + 1], ry)
            plsc.store_scatter(relbuf, [pos + 2], rz)
            plsc.store_scatter(relbuf, [pos + 3], vcf)
            plsc.store_scatter(relbuf, [pos + 4], zeros16)
            plsc.store_scatter(relbuf, [pos + 5], zeros16)
            plsc.store_scatter(relbuf, [pos + 6], zeros16)
            plsc.store_scatter(relbuf, [pos + 7], zeros16)

        cp0.wait()
        cp1.wait()
        pltpu.sync_copy(rows_a, embg_h.at[pl.ds(base, half)])
        pltpu.sync_copy(rows_b, embg_h.at[pl.ds(base + half, half)])
        pltpu.sync_copy(relbuf, relm_h.at[pl.ds(base * 8, r * 8)])

    return k(nni, minsq, colmin, ptsf, validf, embf)


def _loss_body(d, emb_i_ref, embg_ref, relm_ref, w1_ref, b1_ref, g_ref,
               beta_ref, w2_ref, b2_ref, sd_ref, sm_ref):
    first = (pl.program_id(0) == 0) & (pl.program_id(1) == 0)
    x = emb_i_ref[0, 0]
    eg = embg_ref[0]
    rel = relm_ref[0]
    w1 = w1_ref[...]
    rel3 = rel[:, 0:3]
    m = rel[:, 3:4]
    norm = jnp.sqrt(jnp.sum(rel3 * rel3, axis=1, keepdims=True))
    dir3 = rel3 / jnp.maximum(norm, 1e-6)
    h = (jnp.dot(x, w1[0:d], preferred_element_type=jnp.float32)
         + jnp.dot(rel3, w1[d:d + 3], preferred_element_type=jnp.float32)
         + jnp.dot(dir3, w1[d + 3:d + 6], preferred_element_type=jnp.float32)
         + b1_ref[...])
    mu = jnp.mean(h, axis=1, keepdims=True)
    var = jnp.mean((h - mu) ** 2, axis=1, keepdims=True)
    hn = (h - mu) / jnp.sqrt(var + 1e-5) * g_ref[...] + beta_ref[...]
    ge = 0.5 * hn * (1.0 + lax.erf(hn / np.float32(np.sqrt(2.0))))
    t = jnp.dot(ge, w2_ref[...], preferred_element_type=jnp.float32) + b2_ref[...]
    dsq = (t - eg) ** 2
    sd = jnp.sum(dsq * m)
    sm = jnp.sum(m)
    sd_ref[...] = jnp.where(first, 0.0, sd_ref[...]) + sd
    sm_ref[...] = jnp.where(first, 0.0, sm_ref[...]) + sm


def _transform_loss(emb4, vi_ix, embg, relm, w1, b1r, gr, betar, w2, b2r):
    bsz, _, n, d = emb4.shape
    bt = _BT
    nt = n // bt
    return pl.pallas_call(
        functools.partial(_loss_body, d),
        grid=(bsz, nt),
        in_specs=[
            pl.BlockSpec((1, 1, bt, d), lambda b, t: (b, vi_ix, t, 0)),
            pl.BlockSpec((1, bt, d), lambda b, t: (b, t, 0)),
            pl.BlockSpec((1, bt, 8), lambda b, t: (b, t, 0)),
            pl.BlockSpec((d + 6, d), lambda b, t: (0, 0)),
            pl.BlockSpec((1, d), lambda b, t: (0, 0)),
            pl.BlockSpec((1, d), lambda b, t: (0, 0)),
            pl.BlockSpec((1, d), lambda b, t: (0, 0)),
            pl.BlockSpec((d, d), lambda b, t: (0, 0)),
            pl.BlockSpec((1, d), lambda b, t: (0, 0)),
        ],
        out_specs=[
            pl.BlockSpec((1, 1), lambda b, t: (0, 0)),
            pl.BlockSpec((1, 1), lambda b, t: (0, 0)),
        ],
        out_shape=[
            jax.ShapeDtypeStruct((1, 1), jnp.float32),
            jax.ShapeDtypeStruct((1, 1), jnp.float32),
        ],
        compiler_params=pltpu.CompilerParams(
            dimension_semantics=("arbitrary", "arbitrary")),
    )(emb4, embg, relm, w1, b1r, gr, betar, w2, b2r)


def kernel(embeddings, pointmaps, valid_masks, W1, b1, ln_g, ln_b, W2, b2):
    bsz, v, n, d = embeddings.shape
    f32 = jnp.float32
    pts = pointmaps.astype(f32)
    valid_f = valid_masks.astype(f32)
    b1r = b1.reshape(1, d)
    gr = ln_g.reshape(1, d)
    betar = ln_b.reshape(1, d)
    b2r = b2.reshape(1, d)

    bn = bsz * n
    total = jnp.float32(0.0)
    npairs = 0
    for i in range(v):
        for j in range(i + 1, v):
            nni3, minsq3, colmin2 = _mutual_nn(
                pts[:, i], jnp.swapaxes(pts[:, j], 1, 2))
            embg, relm = _sc_corr(
                nni3.reshape(bn), minsq3.reshape(bn), colmin2.reshape(bn),
                pts.reshape(bsz * v * n * 3), valid_f.reshape(bsz * v * n),
                embeddings.reshape(bsz * v * n, d), n, bsz, v, i, j)
            sdb, smb = _transform_loss(
                embeddings, i, embg.reshape(bsz, n, d),
                relm.reshape(bsz, n, 8), W1, b1r, gr, betar, W2, b2r)
            total = total + sdb[0, 0] / (smb[0, 0] * d + 1e-6)
            npairs += 1
    return total / npairs
